# baseline (speedup 1.0000x reference)
import functools
import jax
import jax.numpy as jnp
from jax import lax
from jax.experimental import pallas as pl
from jax.experimental.pallas import tpu as pltpu
from jax.experimental.pallas import tpu_sc as plsc

B, T = 8, 4096
V, D = 8192, 256
NW = 32
TOK_PER_W = (B * T) // NW
W = 128
CHUNKS = TOK_PER_W // W
L = 16
TILES_PER_B = T // TOK_PER_W

_mesh = plsc.VectorSubcoreMesh(core_axis_name="c", subcore_axis_name="s")


@functools.partial(
    pl.kernel,
    mesh=_mesh,
    out_type=jax.ShapeDtypeStruct((B, D, T), jnp.float32),
    scratch_types=[
        pltpu.VMEM((TOK_PER_W,), jnp.int32),
        pltpu.VMEM((W, D), jnp.float32),
        pltpu.VMEM((D, W), jnp.float32),
        pltpu.SemaphoreType.DMA,
    ],
    compiler_params=pltpu.CompilerParams(
        use_tc_tiling_on_sc=False, needs_layout_passes=False
    ),
)
def _vq_decode(tokens_hbm, embed_hbm, out_hbm, idx_v, rows_v, outt_v, sem):
    cid = lax.axis_index("c")
    sid = lax.axis_index("s")
    wid = sid * 2 + cid
    b = wid // TILES_PER_B
    t_base = (wid % TILES_PER_B) * TOK_PER_W

    pltpu.sync_copy(tokens_hbm.at[b, pl.ds(t_base, TOK_PER_W)], idx_v)

    t_iota = lax.iota(jnp.int32, L)

    for c in range(CHUNKS):
        pltpu.async_copy(
            embed_hbm.at[idx_v.at[pl.ds(c * W, W)]], rows_v, sem
        ).wait()

        def body(d, _):
            d_vec = jnp.full((L,), 0, jnp.int32) + d
            for tb in range(W // L):
                t_vec = t_iota + (tb * L)
                vals = plsc.load_gather(rows_v, [t_vec, d_vec])
                plsc.store_scatter(outt_v, [d_vec, t_vec], vals)
            return 0

        lax.fori_loop(0, D, body, 0)

        pltpu.sync_copy(outt_v, out_hbm.at[b, :, pl.ds(t_base + c * W, W)])


def kernel(tokens, embed):
    return _vq_decode(tokens, embed)


# speedup vs baseline: 1.5517x; 1.5517x over previous
import functools
import jax
import jax.numpy as jnp
from jax import lax
from jax.experimental import pallas as pl
from jax.experimental.pallas import tpu as pltpu
from jax.experimental.pallas import tpu_sc as plsc

B, T = 8, 4096
V, D = 8192, 256
NW = 32
TOK_PER_W = (B * T) // NW
W = 64
CHUNKS = TOK_PER_W // W
L = 16
TILES_PER_B = T // TOK_PER_W

_mesh = plsc.VectorSubcoreMesh(core_axis_name="c", subcore_axis_name="s")


@functools.partial(
    pl.kernel,
    mesh=_mesh,
    out_type=jax.ShapeDtypeStruct((B, D, T), jnp.float32),
    scratch_types=[
        pltpu.VMEM((TOK_PER_W,), jnp.int32),
        pltpu.VMEM((2, W, D), jnp.float32),
        pltpu.VMEM((2, D, W), jnp.float32),
        pltpu.SemaphoreType.DMA((2,)),
        pltpu.SemaphoreType.DMA((2,)),
    ],
    compiler_params=pltpu.CompilerParams(
        use_tc_tiling_on_sc=False, needs_layout_passes=False
    ),
)
def _vq_decode(tokens_hbm, embed_hbm, out_hbm, idx_v, rows_v, outt_v, gsem, osem):
    cid = lax.axis_index("c")
    sid = lax.axis_index("s")
    wid = sid * 2 + cid
    b = wid // TILES_PER_B
    t_base = (wid % TILES_PER_B) * TOK_PER_W

    pltpu.sync_copy(tokens_hbm.at[b, pl.ds(t_base, TOK_PER_W)], idx_v)

    d_iota = lax.iota(jnp.int32, L)

    def start_gather(c):
        bi = c % 2
        return pltpu.async_copy(
            embed_hbm.at[idx_v.at[pl.ds(c * W, W)]],
            rows_v.at[bi],
            gsem.at[bi],
        )

    gathers = [None] * CHUNKS
    writes = [None] * CHUNKS
    gathers[0] = start_gather(0)

    for c in range(CHUNKS):
        bi = c % 2
        if c + 1 < CHUNKS:
            gathers[c + 1] = start_gather(c + 1)
        gathers[c].wait()
        if c >= 2:
            writes[c - 2].wait()

        rows = rows_v.at[bi]
        outt = outt_v.at[bi]

        @plsc.parallel_loop(0, W, unroll=2)
        def _(t):
            t_vec = jnp.full((L,), 0, jnp.int32) + t
            for db in range(D // L):
                vals = rows[t, pl.ds(db * L, L)]
                plsc.store_scatter(outt, [d_iota + db * L, t_vec], vals)

        writes[c] = pltpu.async_copy(
            outt, out_hbm.at[b, :, pl.ds(t_base + c * W, W)], osem.at[bi]
        )

    writes[CHUNKS - 2].wait()
    writes[CHUNKS - 1].wait()


def kernel(tokens, embed):
    return _vq_decode(tokens, embed)


# speedup vs baseline: 2.7877x; 1.7965x over previous
import functools
import jax
import jax.numpy as jnp
from jax import lax
from jax.experimental import pallas as pl
from jax.experimental.pallas import tpu as pltpu
from jax.experimental.pallas import tpu_sc as plsc

B, T = 8, 4096
V, D = 8192, 256
NW = 32
TOK_PER_W = (B * T) // NW
W = 64
CHUNKS = TOK_PER_W // W
L = 16
TILES_PER_B = T // TOK_PER_W

_mesh = plsc.VectorSubcoreMesh(core_axis_name="c", subcore_axis_name="s")


@functools.partial(
    pl.kernel,
    mesh=_mesh,
    out_type=jax.ShapeDtypeStruct((B, D, T), jnp.float32),
    scratch_types=[
        pltpu.VMEM((TOK_PER_W,), jnp.int32),
        pltpu.VMEM((2, W, D), jnp.float32),
        pltpu.VMEM((2, D, W + 1), jnp.float32),
        pltpu.SemaphoreType.DMA((2,)),
        pltpu.SemaphoreType.DMA((2,)),
    ],
    compiler_params=pltpu.CompilerParams(
        use_tc_tiling_on_sc=False, needs_layout_passes=False
    ),
)
def _vq_decode(tokens_hbm, embed_hbm, out_hbm, idx_v, rows_v, outt_v, gsem, osem):
    cid = lax.axis_index("c")
    sid = lax.axis_index("s")
    wid = sid * 2 + cid
    b = wid // TILES_PER_B
    t_base = (wid % TILES_PER_B) * TOK_PER_W

    pltpu.sync_copy(tokens_hbm.at[b, pl.ds(t_base, TOK_PER_W)], idx_v)

    d_iota = lax.iota(jnp.int32, L)

    def start_gather(c):
        bi = c % 2
        return pltpu.async_copy(
            embed_hbm.at[idx_v.at[pl.ds(c * W, W)]],
            rows_v.at[bi],
            gsem.at[bi],
        )

    gathers = [None] * CHUNKS
    writes = [None] * CHUNKS
    gathers[0] = start_gather(0)

    for c in range(CHUNKS):
        bi = c % 2
        if c + 1 < CHUNKS:
            gathers[c + 1] = start_gather(c + 1)
        gathers[c].wait()
        if c >= 2:
            writes[c - 2].wait()

        rows = rows_v.at[bi]
        outt = outt_v.at[bi]

        @plsc.parallel_loop(0, W, unroll=4)
        def _(t):
            t_vec = jnp.full((L,), 0, jnp.int32) + t
            for db in range(D // L):
                vals = rows[t, pl.ds(db * L, L)]
                plsc.store_scatter(outt, [d_iota + db * L, t_vec], vals)

        writes[c] = pltpu.async_copy(
            outt.at[:, pl.ds(0, W)],
            out_hbm.at[b, :, pl.ds(t_base + c * W, W)],
            osem.at[bi],
        )

    writes[CHUNKS - 2].wait()
    writes[CHUNKS - 1].wait()


def kernel(tokens, embed):
    return _vq_decode(tokens, embed)


# speedup vs baseline: 4.2437x; 1.5223x over previous
import functools
import jax
import jax.numpy as jnp
from jax import lax
from jax.experimental import pallas as pl
from jax.experimental.pallas import tpu as pltpu
from jax.experimental.pallas import tpu_sc as plsc

B, T = 8, 4096
V, D = 8192, 256
NW = 32
TOK_PER_W = (B * T) // NW
W = 64
CHUNKS = TOK_PER_W // W
L = 16
TILES_PER_B = T // TOK_PER_W
DB = D // 8
TB = T // 128

_mesh = plsc.VectorSubcoreMesh(core_axis_name="c", subcore_axis_name="s")


@functools.partial(
    pl.kernel,
    mesh=_mesh,
    out_type=jax.ShapeDtypeStruct((B, DB, TB, 8, 128), jnp.float32),
    scratch_types=[
        pltpu.VMEM((TOK_PER_W,), jnp.int32),
        pltpu.VMEM((2, W, D), jnp.float32),
        pltpu.VMEM((2, DB, 8, W + 1), jnp.float32),
        pltpu.SemaphoreType.DMA((2,)),
        pltpu.SemaphoreType.DMA((2,)),
    ],
    compiler_params=pltpu.CompilerParams(
        use_tc_tiling_on_sc=False, needs_layout_passes=False
    ),
)
def _vq_decode(tokens_hbm, embed_hbm, out_hbm, idx_v, rows_v, outt_v, gsem, osem):
    cid = lax.axis_index("c")
    sid = lax.axis_index("s")
    wid = sid * 2 + cid
    b = wid // TILES_PER_B
    t_base = (wid % TILES_PER_B) * TOK_PER_W

    pltpu.sync_copy(tokens_hbm.at[b, pl.ds(t_base, TOK_PER_W)], idx_v)

    d_iota = lax.iota(jnp.int32, L)

    def start_gather(c):
        bi = c % 2
        return pltpu.async_copy(
            embed_hbm.at[idx_v.at[pl.ds(c * W, W)]],
            rows_v.at[bi],
            gsem.at[bi],
        )

    gathers = [None] * CHUNKS
    writes = [None] * CHUNKS
    gathers[0] = start_gather(0)

    for c in range(CHUNKS):
        bi = c % 2
        if c + 1 < CHUNKS:
            gathers[c + 1] = start_gather(c + 1)
        gathers[c].wait()
        if c >= 2:
            writes[c - 2].wait()

        rows = rows_v.at[bi]
        outt = outt_v.at[bi]

        @plsc.parallel_loop(0, W, unroll=4)
        def _(t):
            t_vec = jnp.full((L,), 0, jnp.int32) + t
            for db in range(D // L):
                vals = rows[t, pl.ds(db * L, L)]
                plsc.store_scatter(
                    outt,
                    [(d_iota + db * L) >> 3, (d_iota + db * L) & 7, t_vec],
                    vals,
                )

        t0 = t_base + c * W
        writes[c] = pltpu.async_copy(
            outt.at[:, :, pl.ds(0, W)],
            out_hbm.at[b, :, t0 // 128, :, pl.ds(t0 % 128, W)],
            osem.at[bi],
        )

    writes[CHUNKS - 2].wait()
    writes[CHUNKS - 1].wait()


def kernel(tokens, embed):
    out5d = _vq_decode(tokens, embed)
    return out5d.transpose(0, 1, 3, 2, 4).reshape(B, D, T)


# speedup vs baseline: 4.8295x; 1.1380x over previous
import functools
import jax
import jax.numpy as jnp
from jax import lax
from jax.experimental import pallas as pl
from jax.experimental.pallas import tpu as pltpu
from jax.experimental.pallas import tpu_sc as plsc

B, T = 8, 4096
V, D = 8192, 256
NW = 32
TOK_PER_W = (B * T) // NW
W = 64
CHUNKS = TOK_PER_W // W
L = 16
TILES_PER_B = T // TOK_PER_W
DBLK = D // 8
TBLK = T // 128
OSTRIDE = 129

_mesh = plsc.VectorSubcoreMesh(core_axis_name="c", subcore_axis_name="s")


@functools.partial(
    pl.kernel,
    mesh=_mesh,
    out_type=jax.ShapeDtypeStruct((B, DBLK, TBLK, 8, 128), jnp.float32),
    scratch_types=[
        pltpu.VMEM((8, 128), jnp.int32),
        pltpu.VMEM((TOK_PER_W,), jnp.int32),
        pltpu.VMEM((2, 2, W, 128), jnp.float32),
        pltpu.VMEM((2, DBLK, 8, OSTRIDE), jnp.float32),
        pltpu.SemaphoreType.DMA((2,)),
        pltpu.SemaphoreType.DMA((2,)),
    ],
    compiler_params=pltpu.CompilerParams(
        use_tc_tiling_on_sc=False, needs_layout_passes=False
    ),
)
def _vq_decode(tokens_hbm, embed_hbm, out_hbm, m0_v, m1_v, rows_v, outt_v,
               gsem, osem):
    cid = lax.axis_index("c")
    sid = lax.axis_index("s")
    wid = sid * 2 + cid
    b = wid // TILES_PER_B
    j0 = (wid % TILES_PER_B) * (TOK_PER_W // 128)
    t_base = (wid % TILES_PER_B) * TOK_PER_W

    pltpu.sync_copy(tokens_hbm.at[pl.ds(j0, TOK_PER_W // 128), b, :], m0_v)

    iota = lax.iota(jnp.int32, L)

    @plsc.parallel_loop(0, TOK_PER_W // L, unroll=4)
    def _(g):
        jj = g // 8
        l0 = (g % 8) * L
        v = m0_v[jj, pl.ds(l0, L)]
        m0 = ((v >> 3) << 4) | (v & 7)
        m0_v[jj, pl.ds(l0, L)] = m0
        plsc.store_scatter(m1_v, [g * L + iota], m0 | 8)

    def start_gathers(c):
        bi = c % 2
        ga = pltpu.async_copy(
            embed_hbm.at[m0_v.at[c // 2, pl.ds((c % 2) * W, W)]],
            rows_v.at[bi, 0],
            gsem.at[bi],
        )
        gb = pltpu.async_copy(
            embed_hbm.at[m1_v.at[pl.ds(c * W, W)]],
            rows_v.at[bi, 1],
            gsem.at[bi],
        )
        return ga, gb

    gathers = [None] * CHUNKS
    writes = [None] * (CHUNKS // 2)
    gathers[0] = start_gathers(0)

    for c in range(CHUNKS):
        bi = c % 2
        oi = (c // 2) % 2
        if c + 1 < CHUNKS:
            gathers[c + 1] = start_gathers(c + 1)
        gathers[c][0].wait()
        gathers[c][1].wait()
        if c % 2 == 0 and c >= 4:
            writes[c // 2 - 2].wait()

        outt = outt_v.at[oi]
        tcol = (c % 2) * W

        @plsc.parallel_loop(0, W, unroll=4)
        def _(t):
            t_vec = jnp.full((L,), tcol, jnp.int32) + t
            for h in range(2):
                for db in range(8):
                    d_vec = iota + (h * 128 + db * L)
                    vals = rows_v[bi, h, t, pl.ds(db * L, L)]
                    plsc.store_scatter(
                        outt, [d_vec >> 3, d_vec & 7, t_vec], vals
                    )

        if c % 2 == 1:
            p = c // 2
            writes[p] = pltpu.async_copy(
                outt.at[:, :, pl.ds(0, 128)],
                out_hbm.at[b, :, t_base // 128 + p, :, :],
                osem.at[oi],
            )

    writes[CHUNKS // 2 - 2].wait()
    writes[CHUNKS // 2 - 1].wait()


def kernel(tokens, embed):
    tokens_in = tokens.reshape(B, T // 128, 128).transpose(1, 0, 2)
    embed_in = (
        embed.reshape(V // 8, 8, 2, 128).transpose(0, 2, 1, 3).reshape(2 * V, 128)
    )
    out5d = _vq_decode(tokens_in, embed_in)
    return out5d.transpose(0, 1, 3, 2, 4).reshape(B, D, T)


# speedup vs baseline: 5.9748x; 1.2371x over previous
import functools
import jax
import jax.numpy as jnp
from jax import lax
from jax.experimental import pallas as pl
from jax.experimental.pallas import tpu as pltpu
from jax.experimental.pallas import tpu_sc as plsc

B, T = 8, 4096
V, D = 8192, 256
NW = 32
TOK_PER_W = (B * T) // NW
W = 64
PAIRS = TOK_PER_W // (2 * W)
L = 16
TILES_PER_B = T // TOK_PER_W
DBLK = D // 8
TBLK = T // 128
OSTRIDE = 129

_mesh = plsc.VectorSubcoreMesh(core_axis_name="c", subcore_axis_name="s")


@functools.partial(
    pl.kernel,
    mesh=_mesh,
    out_type=jax.ShapeDtypeStruct((B, DBLK, TBLK, 8, 128), jnp.float32),
    scratch_types=[
        pltpu.VMEM((8, 128), jnp.int32),
        pltpu.VMEM((TOK_PER_W,), jnp.int32),
        pltpu.VMEM((2, 2, W, 128), jnp.float32),
        pltpu.VMEM((2, DBLK, 8, OSTRIDE), jnp.float32),
        pltpu.SemaphoreType.DMA((2,)),
        pltpu.SemaphoreType.DMA((2,)),
    ],
    compiler_params=pltpu.CompilerParams(
        use_tc_tiling_on_sc=False, needs_layout_passes=False
    ),
)
def _vq_decode(tokens_hbm, embed_hbm, out_hbm, m0_v, m1_v, rows_v, outt_v,
               gsem, osem):
    cid = lax.axis_index("c")
    sid = lax.axis_index("s")
    wid = sid * 2 + cid
    b = wid // TILES_PER_B
    j0 = (wid % TILES_PER_B) * (TOK_PER_W // 128)
    tb0 = (wid % TILES_PER_B) * (TOK_PER_W // 128)

    pltpu.sync_copy(tokens_hbm.at[pl.ds(j0, TOK_PER_W // 128), b, :], m0_v)

    iota = lax.iota(jnp.int32, L)

    @plsc.parallel_loop(0, TOK_PER_W // L, unroll=4)
    def _(g):
        jj = g // 8
        l0 = (g % 8) * L
        v = m0_v[jj, pl.ds(l0, L)]
        m0 = ((v >> 3) << 4) | (v & 7)
        m0_v[jj, pl.ds(l0, L)] = m0
        plsc.store_scatter(m1_v, [g * L + iota], m0 | 8)

    def issue_gather(p, sub):
        pltpu.async_copy(
            embed_hbm.at[m0_v.at[p, pl.ds(sub * W, W)]],
            rows_v.at[sub, 0],
            gsem.at[sub],
        )
        pltpu.async_copy(
            embed_hbm.at[m1_v.at[pl.ds((2 * p + sub) * W, W)]],
            rows_v.at[sub, 1],
            gsem.at[sub],
        )

    def wait_gather(sub):
        for h in range(2):
            pltpu.make_async_copy(
                embed_hbm.at[pl.ds(0, W), :], rows_v.at[sub, h], gsem.at[sub]
            ).wait()

    def wait_write(oi):
        pltpu.make_async_copy(
            outt_v.at[oi].at[:, :, pl.ds(0, 128)],
            out_hbm.at[b, :, tb0, :, :],
            osem.at[oi],
        ).wait()

    def issue_write(p, oi):
        pltpu.async_copy(
            outt_v.at[oi].at[:, :, pl.ds(0, 128)],
            out_hbm.at[b, :, tb0 + p, :, :],
            osem.at[oi],
        )

    def transpose(sub, outt):
        tcol = sub * W

        @plsc.parallel_loop(0, W, unroll=4)
        def _(t):
            t_vec = jnp.full((L,), tcol, jnp.int32) + t
            for h in range(2):
                for db in range(8):
                    d_vec = iota + (h * 128 + db * L)
                    vals = rows_v[sub, h, t, pl.ds(db * L, L)]
                    plsc.store_scatter(
                        outt, [d_vec >> 3, d_vec & 7, t_vec], vals
                    )

    issue_gather(0, 0)
    issue_gather(0, 1)

    def pair_body(p, _):
        oi = p % 2
        outt = outt_v.at[oi]
        wait_gather(0)

        @pl.when(p >= 2)
        def _():
            wait_write(oi)

        transpose(0, outt)
        issue_gather(p + 1, 0)
        wait_gather(1)
        transpose(1, outt)
        issue_gather(p + 1, 1)
        issue_write(p, oi)
        return 0

    lax.fori_loop(0, PAIRS - 1, pair_body, 0)

    p_last = PAIRS - 1
    oi = p_last % 2
    outt = outt_v.at[oi]
    wait_gather(0)
    wait_write(oi)
    transpose(0, outt)
    wait_gather(1)
    transpose(1, outt)
    issue_write(p_last, oi)

    wait_write(1 - oi)
    wait_write(oi)


def kernel(tokens, embed):
    tokens_in = tokens.reshape(B, T // 128, 128).transpose(1, 0, 2)
    embed_in = (
        embed.reshape(V // 8, 8, 2, 128).transpose(0, 2, 1, 3).reshape(2 * V, 128)
    )
    out5d = _vq_decode(tokens_in, embed_in)
    return out5d.transpose(0, 1, 3, 2, 4).reshape(B, D, T)
